# Initial kernel scaffold; baseline (speedup 1.0000x reference)
import jax
import jax.numpy as jnp
from jax.experimental import pallas as pl


def kernel(hidden_states, gate_w, w_gate, w_up, w_down, s_gate, s_up, s_down):
    raise NotImplementedError("write your pallas kernel here")


# speedup vs baseline: 1.5217x; 1.5217x over previous
import jax
import jax.numpy as jnp
from jax.experimental import pallas as pl
from jax.experimental.pallas import tpu as pltpu
import functools

B, S, H = 1, 2048, 768
E = 8
K = 2
F_INT = 1024
S_INT = 512

TOKEN_TILE = 512


def _moe_kernel(x_ref, gate_w_ref, wg_ref, wu_ref, wd_ref,
                sg_ref, su_ref, sd_ref, out_ref, combine_ref):
    e = pl.program_id(1)
    x = x_ref[...]

    @pl.when(e == 0)
    def _():
        logits = jnp.dot(x, gate_w_ref[...], preferred_element_type=jnp.float32)
        m1 = jnp.max(logits, axis=-1, keepdims=True)
        i1 = jnp.argmax(logits, axis=-1)[:, None]
        eids = jax.lax.broadcasted_iota(jnp.int32, logits.shape, 1)
        masked = jnp.where(eids == i1, -jnp.inf, logits)
        m2 = jnp.max(masked, axis=-1, keepdims=True)
        i2 = jnp.argmax(masked, axis=-1)[:, None]
        p1 = 1.0 / (1.0 + jnp.exp(m2 - m1))
        p2 = 1.0 - p1
        combine_ref[...] = jnp.where(eids == i1, p1, 0.0) + jnp.where(eids == i2, p2, 0.0)

        sg = jnp.dot(x, sg_ref[...], preferred_element_type=jnp.float32)
        su = jnp.dot(x, su_ref[...], preferred_element_type=jnp.float32)
        hmid = (sg * jax.lax.logistic(sg)) * su
        out_ref[...] = jnp.dot(hmid, sd_ref[...], preferred_element_type=jnp.float32)

    g = jnp.dot(x, wg_ref[0], preferred_element_type=jnp.float32)
    u = jnp.dot(x, wu_ref[0], preferred_element_type=jnp.float32)
    hmid = (g * jax.lax.logistic(g)) * u
    eo = jnp.dot(hmid, wd_ref[0], preferred_element_type=jnp.float32)
    comb = combine_ref[...]
    cids = jax.lax.broadcasted_iota(jnp.int32, comb.shape, 1)
    w_e = jnp.sum(jnp.where(cids == e, comb, 0.0), axis=1, keepdims=True)
    out_ref[...] += w_e * eo


@jax.jit
def kernel(hidden_states, gate_w, w_gate, w_up, w_down, s_gate, s_up, s_down):
    b, s, h = hidden_states.shape
    x = hidden_states.reshape(-1, h)
    T = x.shape[0]
    n_tiles = T // TOKEN_TILE

    out = pl.pallas_call(
        _moe_kernel,
        grid=(n_tiles, E),
        in_specs=[
            pl.BlockSpec((TOKEN_TILE, H), lambda t, e: (t, 0)),
            pl.BlockSpec((H, E), lambda t, e: (0, 0)),
            pl.BlockSpec((1, H, F_INT), lambda t, e: (e, 0, 0)),
            pl.BlockSpec((1, H, F_INT), lambda t, e: (e, 0, 0)),
            pl.BlockSpec((1, F_INT, H), lambda t, e: (e, 0, 0)),
            pl.BlockSpec((H, S_INT), lambda t, e: (0, 0)),
            pl.BlockSpec((H, S_INT), lambda t, e: (0, 0)),
            pl.BlockSpec((S_INT, H), lambda t, e: (0, 0)),
        ],
        out_specs=pl.BlockSpec((TOKEN_TILE, H), lambda t, e: (t, 0)),
        out_shape=jax.ShapeDtypeStruct((T, H), jnp.float32),
        scratch_shapes=[pltpu.VMEM((TOKEN_TILE, E), jnp.float32)],
    )(x, gate_w, w_gate, w_up, w_down, s_gate, s_up, s_down)
    return out.reshape(b, s, h)
